# baseline (speedup 1.0000x reference)
import functools

import jax
import jax.numpy as jnp
from jax import lax
from jax.experimental import pallas as pl
from jax.experimental.pallas import tpu as pltpu
from jax.experimental.pallas import tpu_sc as plsc

_B, _H, _GY, _GX = 32, 3, 160, 160
_NTOT = _B * _H * _GY * _GX
_NA = 20000
_NC, _NS = 2, 16
_NW = _NC * _NS
_OWN = _NTOT // _NW
_CHUNK = 128

_mesh = plsc.VectorSubcoreMesh(core_axis_name="c", subcore_axis_name="s")


@functools.partial(
    pl.kernel,
    mesh=_mesh,
    compiler_params=pltpu.CompilerParams(needs_layout_passes=False),
    out_type=jax.ShapeDtypeStruct((_NW * 16,), jnp.float32),
    scratch_types=[
        pltpu.VMEM((_NA,), jnp.int32),
        pltpu.VMEM((_OWN,), jnp.int32),
        pltpu.VMEM((_NA + _CHUNK,), jnp.int32),
        pltpu.VMEM((_CHUNK,), jnp.float32),
        pltpu.VMEM((16,), jnp.float32),
        pltpu.SemaphoreType.DMA,
    ],
)
def _sc_masked_sum(x_hbm, idx_hbm, out_hbm, idx_v, table_v, compact_v,
                   xbuf_v, acc_v, sem):
    wid = lax.axis_index("s") * _NC + lax.axis_index("c")
    base = wid * _OWN
    pltpu.sync_copy(idx_hbm, idx_v)

    def _slot_group(g):
        s = pl.multiple_of(g * 16, 16)
        idx16 = idx_v[pl.ds(s, 16)]
        pos = g * 16 + lax.iota(jnp.int32, 16)
        rel = idx16 - base
        m = (rel >= 0) & (rel < _OWN)
        relc = jnp.clip(rel, 0, _OWN - 1)
        return idx16, pos, m, relc

    def _pass1(g, c):
        idx16, pos, m, relc = _slot_group(g)
        plsc.store_scatter(table_v, [relc], pos, mask=m)
        return c

    lax.fori_loop(0, _NA // 16, _pass1, jnp.int32(0))

    def _pass2(g, off):
        idx16, pos, m, relc = _slot_group(g)
        w = plsc.load_gather(table_v, [relc], mask=m)
        win = m & (w == pos)
        wi = win.astype(jnp.int32)
        cs = plsc.cumsum(wi)
        addr = off + cs - wi
        plsc.store_scatter(compact_v, [addr], idx16, mask=win)
        return off + jnp.sum(wi)

    cnt = lax.fori_loop(0, _NA // 16, _pass2, jnp.int32(0))

    for g in range(_CHUNK // 16):
        compact_v[pl.ds(cnt + g * 16, 16)] = jnp.zeros((16,), jnp.int32)

    def _pass3(j, acc):
        s2 = j * _CHUNK
        pltpu.async_copy(
            x_hbm.at[compact_v.at[pl.ds(s2, _CHUNK)]], xbuf_v, sem
        ).wait()
        for g in range(_CHUNK // 16):
            v16 = xbuf_v[pl.ds(g * 16, 16)]
            lane = s2 + g * 16 + lax.iota(jnp.int32, 16)
            acc = acc + jnp.where(lane < cnt, v16, 0.0)
        return acc

    nch = (cnt + _CHUNK - 1) // _CHUNK
    acc = lax.fori_loop(0, nch, _pass3, jnp.zeros((16,), jnp.float32))
    acc_v[...] = acc
    pltpu.sync_copy(acc_v, out_hbm.at[pl.ds(wid * 16, 16)])


def _tc_body(x_ref, out_ref):
    @pl.when(pl.program_id(0) == 0)
    def _init():
        out_ref[0, 0] = 0.0

    x = x_ref[...]
    f = jnp.maximum(x, 0.0) + jnp.log1p(jnp.exp(-jnp.abs(x)))
    out_ref[0, 0] += jnp.sum(f)


_TC_GRID = 8
_ROWS = _NTOT // 128

_tc_softplus_sum = pl.pallas_call(
    _tc_body,
    grid=(_TC_GRID,),
    in_specs=[pl.BlockSpec((_ROWS // _TC_GRID, 128), lambda i: (i, 0))],
    out_specs=pl.BlockSpec((1, 1), lambda i: (0, 0), memory_space=pltpu.SMEM),
    out_shape=jax.ShapeDtypeStruct((1, 1), jnp.float32),
)


def kernel(pre_activation_o, img_idxs, head_idxs, grid_y_idxs, grid_x_idxs):
    flat = (
        (img_idxs.astype(jnp.int32) * _H + head_idxs) * _GY + grid_y_idxs
    ) * _GX + grid_x_idxs
    partials = _sc_masked_sum(pre_activation_o.reshape(_NTOT), flat)
    dense = _tc_softplus_sum(pre_activation_o.reshape(_ROWS, 128))[0, 0]
    return (dense - jnp.sum(partials)) / _NTOT


# speedup vs baseline: 1.1775x; 1.1775x over previous
import functools

import jax
import jax.numpy as jnp
from jax import lax
from jax.experimental import pallas as pl
from jax.experimental.pallas import tpu as pltpu
from jax.experimental.pallas import tpu_sc as plsc

_B, _H, _GY, _GX = 32, 3, 160, 160
_NTOT = _B * _H * _GY * _GX
_NA = 20000
_NC, _NS = 2, 16
_NW = _NC * _NS
_OWN = _NTOT // _NW
_HALF = _OWN // 2
_GRP = _NA // 16
_UNROLL = 10

_mesh = plsc.VectorSubcoreMesh(core_axis_name="c", subcore_axis_name="s")


@functools.partial(
    pl.kernel,
    mesh=_mesh,
    compiler_params=pltpu.CompilerParams(needs_layout_passes=False),
    out_type=jax.ShapeDtypeStruct((_NW * 16,), jnp.float32),
    scratch_types=[
        pltpu.VMEM((_NA,), jnp.int32),
        pltpu.VMEM((_HALF,), jnp.int32),
        pltpu.VMEM((_HALF,), jnp.float32),
        pltpu.VMEM((16,), jnp.float32),
    ],
)
def _sc_masked_sum(x_hbm, idx_hbm, out_hbm, idx_v, table_v, strip_v, acc_v):
    wid = lax.axis_index("s") * _NC + lax.axis_index("c")
    base = wid * _OWN
    pltpu.sync_copy(idx_hbm, idx_v)

    def _slot_group(g, hbase):
        s = pl.multiple_of(g * 16, 16)
        idx16 = idx_v[pl.ds(s, 16)]
        pos = g * 16 + lax.iota(jnp.int32, 16)
        rel = idx16 - hbase
        m = (rel >= 0) & (rel < _HALF)
        relc = jnp.clip(rel, 0, _HALF - 1)
        return pos, m, relc

    acc = jnp.zeros((16,), jnp.float32)
    for h in range(2):
        hbase = base + h * _HALF
        pltpu.sync_copy(x_hbm.at[pl.ds(hbase, _HALF)], strip_v)

        def _pass1(i, c, hbase=hbase):
            for k in range(_UNROLL):
                g = i * _UNROLL + k
                pos, m, relc = _slot_group(g, hbase)
                plsc.store_scatter(table_v, [relc], pos, mask=m)
            return c

        lax.fori_loop(0, _GRP // _UNROLL, _pass1, jnp.int32(0))

        def _pass2(i, a, hbase=hbase):
            for k in range(_UNROLL):
                g = i * _UNROLL + k
                pos, m, relc = _slot_group(g, hbase)
                w = plsc.load_gather(table_v, [relc], mask=m)
                win = m & (w == pos)
                v = plsc.load_gather(strip_v, [relc], mask=win)
                a = a + jnp.where(win, v, 0.0)
            return a

        acc = lax.fori_loop(0, _GRP // _UNROLL, _pass2, acc)

    acc_v[...] = acc
    pltpu.sync_copy(acc_v, out_hbm.at[pl.ds(wid * 16, 16)])


def _tc_body(x_ref, out_ref):
    @pl.when(pl.program_id(0) == 0)
    def _init():
        out_ref[0, 0] = 0.0

    x = x_ref[...]
    f = jnp.maximum(x, 0.0) + jnp.log1p(jnp.exp(-jnp.abs(x)))
    out_ref[0, 0] += jnp.sum(f)


_TC_GRID = 8
_ROWS = _NTOT // 128

_tc_softplus_sum = pl.pallas_call(
    _tc_body,
    grid=(_TC_GRID,),
    in_specs=[pl.BlockSpec((_ROWS // _TC_GRID, 128), lambda i: (i, 0))],
    out_specs=pl.BlockSpec((1, 1), lambda i: (0, 0), memory_space=pltpu.SMEM),
    out_shape=jax.ShapeDtypeStruct((1, 1), jnp.float32),
)


def kernel(pre_activation_o, img_idxs, head_idxs, grid_y_idxs, grid_x_idxs):
    flat = (
        (img_idxs.astype(jnp.int32) * _H + head_idxs) * _GY + grid_y_idxs
    ) * _GX + grid_x_idxs
    partials = _sc_masked_sum(pre_activation_o.reshape(_NTOT), flat)
    dense = _tc_softplus_sum(pre_activation_o.reshape(_ROWS, 128))[0, 0]
    return (dense - jnp.sum(partials)) / _NTOT


# speedup vs baseline: 1.1923x; 1.0126x over previous
import functools

import jax
import jax.numpy as jnp
from jax import lax
from jax.experimental import pallas as pl
from jax.experimental.pallas import tpu as pltpu
from jax.experimental.pallas import tpu_sc as plsc

_B, _H, _GY, _GX = 32, 3, 160, 160
_NTOT = _B * _H * _GY * _GX
_NA = 20000
_NC, _NS = 2, 16
_NW = _NC * _NS
_OWN = _NTOT // _NW
_HALF = _OWN // 2
_GRP = _NA // 16
_UNROLL = 10

_mesh = plsc.VectorSubcoreMesh(core_axis_name="c", subcore_axis_name="s")


@functools.partial(
    pl.kernel,
    mesh=_mesh,
    compiler_params=pltpu.CompilerParams(needs_layout_passes=False),
    out_type=jax.ShapeDtypeStruct((_NW * 16,), jnp.float32),
    scratch_types=[
        pltpu.VMEM((_NA,), jnp.int32),
        pltpu.VMEM((_HALF,), jnp.int32),
        pltpu.VMEM((_HALF,), jnp.float32),
        pltpu.VMEM((16,), jnp.float32),
        pltpu.SemaphoreType.DMA,
    ],
)
def _sc_masked_sum(x_hbm, idx_hbm, out_hbm, idx_v, table_v, strip_v, acc_v,
                   sem):
    wid = lax.axis_index("s") * _NC + lax.axis_index("c")
    base = wid * _OWN
    pltpu.sync_copy(idx_hbm, idx_v)
    neg1 = jnp.full((16,), -1, jnp.int32)

    acc = jnp.zeros((16,), jnp.float32)
    for h in range(2):
        hbase = base + h * _HALF
        strip_dma = pltpu.async_copy(
            x_hbm.at[pl.ds(hbase, _HALF)], strip_v, sem
        )

        def _init(i, c):
            for k in range(_UNROLL):
                g = i * _UNROLL + k
                table_v[pl.ds(pl.multiple_of(g * 16, 16), 16)] = neg1
            return c

        lax.fori_loop(0, _HALF // 16 // _UNROLL, _init, jnp.int32(0))
        strip_dma.wait()

        def _sweep(i, a, hbase=hbase):
            for k in range(_UNROLL):
                g = i * _UNROLL + k
                s = pl.multiple_of(g * 16, 16)
                idx16 = idx_v[pl.ds(s, 16)]
                pos = g * 16 + lax.iota(jnp.int32, 16)
                rel = idx16 - hbase
                m = (rel >= 0) & (rel < _HALF)
                relc = jnp.where(m, rel, 0)
                old = plsc.load_gather(table_v, [relc], mask=m)
                plsc.store_scatter(table_v, [relc], pos, mask=m)
                w = plsc.load_gather(table_v, [relc], mask=m)
                first = (m & (old == neg1)) & (w == pos)
                v = plsc.load_gather(strip_v, [relc], mask=first)
                a = a + jnp.where(first, v, 0.0)
            return a

        acc = lax.fori_loop(0, _GRP // _UNROLL, _sweep, acc)

    acc_v[...] = acc
    pltpu.sync_copy(acc_v, out_hbm.at[pl.ds(wid * 16, 16)])


def _tc_body(x_ref, out_ref):
    @pl.when(pl.program_id(0) == 0)
    def _init():
        out_ref[0, 0] = 0.0

    x = x_ref[...]
    f = jnp.maximum(x, 0.0) + jnp.log1p(jnp.exp(-jnp.abs(x)))
    out_ref[0, 0] += jnp.sum(f)


_TC_GRID = 8
_ROWS = _NTOT // 128

_tc_softplus_sum = pl.pallas_call(
    _tc_body,
    grid=(_TC_GRID,),
    in_specs=[pl.BlockSpec((_ROWS // _TC_GRID, 128), lambda i: (i, 0))],
    out_specs=pl.BlockSpec((1, 1), lambda i: (0, 0), memory_space=pltpu.SMEM),
    out_shape=jax.ShapeDtypeStruct((1, 1), jnp.float32),
)


def kernel(pre_activation_o, img_idxs, head_idxs, grid_y_idxs, grid_x_idxs):
    flat = (
        (img_idxs.astype(jnp.int32) * _H + head_idxs) * _GY + grid_y_idxs
    ) * _GX + grid_x_idxs
    partials = _sc_masked_sum(pre_activation_o.reshape(_NTOT), flat)
    dense = _tc_softplus_sum(pre_activation_o.reshape(_ROWS, 128))[0, 0]
    return (dense - jnp.sum(partials)) / _NTOT
